# baseline (speedup 1.0000x reference)
import functools

import jax
import jax.numpy as jnp
from jax import lax
from jax.experimental import pallas as pl
from jax.experimental.pallas import tpu as pltpu
from jax.experimental.pallas import tpu_sc as plsc

NUM_GROUP_K = 512
GROUP_SIZE_K = 32
ROW_PAD = 16
GBLK = 64
NCHUNK = 32
CAP = 768
RCH = 8


def _fps_body(x_ref, y_ref, z_ref, cx_ref, cy_ref, cz_ref, dist_ref):
    B, N = x_ref.shape
    G = cx_ref.shape[1]
    x = x_ref[...]
    y = y_ref[...]
    z = z_ref[...]
    flane = lax.broadcasted_iota(jnp.int32, (B, N), 1).astype(jnp.float32)
    gcol = lax.broadcasted_iota(jnp.int32, (B, G), 1)
    bigf = jnp.float32(2.0 * N)

    dist_ref[...] = jnp.full((B, N), jnp.inf, dtype=jnp.float32)
    lx0 = x[:, 0:1]
    ly0 = y[:, 0:1]
    lz0 = z[:, 0:1]
    cx0 = jnp.where(gcol == 0, lx0, 0.0)
    cy0 = jnp.where(gcol == 0, ly0, 0.0)
    cz0 = jnp.where(gcol == 0, lz0, 0.0)

    def step(j, carry):
        lx, ly, lz, cx, cy, cz = carry
        dx = x - lx
        dy = y - ly
        dz = z - lz
        d = (dx * dx + dy * dy) + dz * dz
        dist = jnp.minimum(dist_ref[...], d)
        dist_ref[...] = dist
        mx = jnp.max(dist, axis=1, keepdims=True)
        nxt = jnp.min(jnp.where(dist == mx, flane, bigf), axis=1, keepdims=True)
        sel = flane == nxt
        lx = jnp.sum(jnp.where(sel, x, 0.0), axis=1, keepdims=True)
        ly = jnp.sum(jnp.where(sel, y, 0.0), axis=1, keepdims=True)
        lz = jnp.sum(jnp.where(sel, z, 0.0), axis=1, keepdims=True)
        hit = gcol == j
        cx = cx + jnp.where(hit, lx, 0.0)
        cy = cy + jnp.where(hit, ly, 0.0)
        cz = cz + jnp.where(hit, lz, 0.0)
        return lx, ly, lz, cx, cy, cz

    _, _, _, cx, cy, cz = lax.fori_loop(
        1, G, step, (lx0, ly0, lz0, cx0, cy0, cz0))
    cx_ref[...] = cx
    cy_ref[...] = cy
    cz_ref[...] = cz


def _fps_centers(x, y, z):
    B, N = x.shape
    G = NUM_GROUP_K
    out = jax.ShapeDtypeStruct((B, G), jnp.float32)
    return pl.pallas_call(
        _fps_body,
        out_shape=(out, out, out),
        scratch_shapes=[pltpu.VMEM((B, N), jnp.float32)],
    )(x, y, z)


def _dist_body(x_ref, y_ref, z_ref, c_ref, d_ref, t_ref):
    N = x_ref.shape[2]
    x = x_ref[0]
    y = y_ref[0]
    z = z_ref[0]
    c = c_ref[0]
    dx = c[:, 0:1] - x
    dy = c[:, 1:2] - y
    dz = c[:, 2:3] - z
    d0 = (dx * dx + dy * dy) + dz * dz
    d_ref[0] = d0
    cm = d0[:, 0:128]
    for ci in range(1, N // 128):
        cm = jnp.minimum(cm, d0[:, ci * 128:(ci + 1) * 128])
    cm = jnp.minimum(cm[:, 0:64], cm[:, 64:128])
    cm = jnp.minimum(cm[:, 0:NCHUNK], cm[:, NCHUNK:64])
    t_ref[0] = jnp.max(cm, axis=1, keepdims=True)


def _knn_dist(x3, y3, z3, center):
    B = x3.shape[0]
    N = x3.shape[2]
    G = NUM_GROUP_K
    grid = (B, G // GBLK)
    return pl.pallas_call(
        _dist_body,
        grid=grid,
        in_specs=[
            pl.BlockSpec((1, 1, N), lambda b, g: (b, 0, 0)),
            pl.BlockSpec((1, 1, N), lambda b, g: (b, 0, 0)),
            pl.BlockSpec((1, 1, N), lambda b, g: (b, 0, 0)),
            pl.BlockSpec((1, GBLK, 3), lambda b, g: (b, g, 0)),
        ],
        out_specs=(
            pl.BlockSpec((1, GBLK, N), lambda b, g: (b, g, 0)),
            pl.BlockSpec((1, GBLK, 1), lambda b, g: (b, g, 0)),
        ),
        out_shape=(
            jax.ShapeDtypeStruct((B, G, N), jnp.float32),
            jax.ShapeDtypeStruct((B, G, 1), jnp.float32),
        ),
    )(x3, y3, z3, center)


def _take16(x, idx):
    dn = lax.GatherDimensionNumbers(
        offset_dims=(), collapsed_slice_dims=(0,), start_index_map=(0,))
    return lax.gather(x, idx[:, None], dn, slice_sizes=(1,),
                      mode=lax.GatherScatterMode.PROMISE_IN_BOUNDS)


def _sc_compress(dflat, tflat, R2, N):
    info = plsc.get_sparse_core_info()
    nw = info.num_cores * info.num_subcores
    rpw = R2 // nw
    mesh = plsc.VectorSubcoreMesh(core_axis_name="c", subcore_axis_name="s")
    capb = CAP + 16

    @functools.partial(
        pl.kernel,
        mesh=mesh,
        compiler_params=pltpu.CompilerParams(use_tc_tiling_on_sc=False,
                                             needs_layout_passes=False),
        out_type=(
            jax.ShapeDtypeStruct((R2, CAP), jnp.float32),
            jax.ShapeDtypeStruct((R2, CAP), jnp.float32),
        ),
        scratch_types=[
            pltpu.VMEM((RCH * N,), jnp.float32),
            pltpu.VMEM((capb,), jnp.float32),
            pltpu.VMEM((capb,), jnp.float32),
            pltpu.VMEM((rpw * 16,), jnp.float32),
            pltpu.VMEM((16,), jnp.int32),
        ],
    )
    def k(d_hbm, t_hbm, cd_hbm, ci_hbm, rows_v, cd_v, ci_v, t_v, ptr_v):
        wid = lax.axis_index("s") * info.num_cores + lax.axis_index("c")
        rbase = wid * rpw
        pltpu.sync_copy(t_hbm.at[pl.ds(rbase * 16, rpw * 16)], t_v)
        iota16 = lax.iota(jnp.int32, 16)
        one16 = jnp.full((16,), 1, jnp.int32)
        zero16 = jnp.zeros((16,), jnp.int32)
        inf16 = jnp.full((16,), jnp.inf, jnp.float32)
        big16 = jnp.full((16,), jnp.float32(3 * N), jnp.float32)
        maxpos = jnp.full((16,), capb - 16, jnp.int32)
        lane15 = jnp.full((16,), 15, jnp.int32)

        def chunk_body(cix, _):
            pltpu.sync_copy(
                d_hbm.at[pl.ds((rbase + cix * RCH) * N, RCH * N)], rows_v)

            def row_body(rl, _):
                def pf(i, _):
                    cd_v[pl.ds(i * 16, 16)] = inf16
                    ci_v[pl.ds(i * 16, 16)] = big16
                    return 0
                lax.fori_loop(0, capb // 16, pf, 0)
                tvec = t_v[pl.ds((cix * RCH + rl) * 16, 16)]

                ptr_v[...] = jnp.zeros((16,), jnp.int32)

                def scan_body(i, _):
                    ptr = ptr_v[...]
                    for u in range(4):
                        iu = i * 4 + u
                        v = rows_v[pl.ds(rl * N + iu * 16, 16)]
                        mask = v <= tvec
                        cs = plsc.cumsum(jnp.where(mask, one16, zero16))
                        pos = jnp.minimum(ptr + cs - 1, maxpos)
                        posf = (iota16 + iu * 16).astype(jnp.float32)
                        plsc.store_scatter(cd_v, [pos], v, mask=mask)
                        plsc.store_scatter(ci_v, [pos], posf, mask=mask)
                        ptr = ptr + _take16(cs, lane15)
                    ptr_v[...] = ptr
                    return 0

                lax.fori_loop(0, N // 64, scan_body, 0)
                r = rbase + cix * RCH + rl
                pltpu.sync_copy(cd_v.at[pl.ds(0, CAP)], cd_hbm.at[r])
                pltpu.sync_copy(ci_v.at[pl.ds(0, CAP)], ci_hbm.at[r])
                return 0

            lax.fori_loop(0, RCH, row_body, 0)
            return 0

        lax.fori_loop(0, rpw // RCH, chunk_body, 0)

    return k(dflat, tflat)


def _sel_body(cd_ref, ci_ref, idx_ref, ds_ref):
    M = idx_ref.shape[2]
    d0 = cd_ref[0]
    idxf = ci_ref[0]
    ds_ref[...] = d0
    mcol = lax.broadcasted_iota(jnp.int32, (GBLK, M), 1)
    bigf = jnp.float32(3.0 * 8192)
    mn0 = jnp.min(d0, axis=1, keepdims=True)

    def body(j, carry):
        acc, mn = carry
        dcur = ds_ref[...]
        am = jnp.min(jnp.where(dcur == mn, idxf, bigf), axis=1, keepdims=True)
        dnew = jnp.where(idxf == am, jnp.inf, dcur)
        ds_ref[...] = dnew
        mn2 = jnp.min(dnew, axis=1, keepdims=True)
        return acc + jnp.where(mcol == j, am.astype(jnp.int32), 0), mn2

    acc, _ = lax.fori_loop(
        0, M, body, (jnp.zeros((GBLK, M), jnp.int32), mn0))
    idx_ref[0] = acc


def _topk_sel(cd3, ci3):
    B = cd3.shape[0]
    G = NUM_GROUP_K
    M = GROUP_SIZE_K
    grid = (B, G // GBLK)
    return pl.pallas_call(
        _sel_body,
        grid=grid,
        in_specs=[
            pl.BlockSpec((1, GBLK, CAP), lambda b, g: (b, g, 0)),
            pl.BlockSpec((1, GBLK, CAP), lambda b, g: (b, g, 0)),
        ],
        out_specs=pl.BlockSpec((1, GBLK, M), lambda b, g: (b, g, 0)),
        out_shape=jax.ShapeDtypeStruct((B, G, M), jnp.int32),
        scratch_shapes=[pltpu.VMEM((GBLK, CAP), jnp.float32)],
    )(cd3, ci3)


def _sc_gather_normalize(flat_idx, pts_pad, cent_pad):
    R = flat_idx.shape[0]
    info = plsc.get_sparse_core_info()
    nw = info.num_cores * info.num_subcores
    rpw = R // nw
    gpw = rpw // GROUP_SIZE_K
    mesh = plsc.VectorSubcoreMesh(core_axis_name="c", subcore_axis_name="s")

    @functools.partial(
        pl.kernel,
        mesh=mesh,
        compiler_params=pltpu.CompilerParams(use_tc_tiling_on_sc=False),
        out_type=jax.ShapeDtypeStruct((R, ROW_PAD), jnp.float32),
        scratch_types=[
            pltpu.VMEM((rpw,), jnp.int32),
            pltpu.VMEM((rpw, ROW_PAD), jnp.float32),
            pltpu.VMEM((gpw, ROW_PAD), jnp.float32),
            pltpu.SemaphoreType.DMA,
        ],
    )
    def k(idx_hbm, pts_hbm, cent_hbm, out_hbm, idx_v, rows_v, cent_v, sem):
        wid = lax.axis_index("s") * info.num_cores + lax.axis_index("c")
        rbase = wid * rpw
        pltpu.sync_copy(idx_hbm.at[pl.ds(rbase, rpw)], idx_v)
        pltpu.async_copy(pts_hbm.at[idx_v], rows_v, sem).wait()
        pltpu.sync_copy(cent_hbm.at[pl.ds(wid * gpw, gpw)], cent_v)

        def body(g, _):
            cvec = cent_v[g]
            base = g * GROUP_SIZE_K
            for j in range(GROUP_SIZE_K):
                rows_v[base + j] = rows_v[base + j] - cvec
            return 0

        lax.fori_loop(0, gpw, body, 0)
        pltpu.sync_copy(rows_v, out_hbm.at[pl.ds(rbase, rpw)])

    return k(flat_idx, pts_pad, cent_pad)


def kernel(pts):
    B, N, C = pts.shape
    G = NUM_GROUP_K
    M = GROUP_SIZE_K
    x = pts[:, :, 0]
    y = pts[:, :, 1]
    z = pts[:, :, 2]
    cx, cy, cz = _fps_centers(x, y, z)
    center = jnp.stack([cx, cy, cz], axis=-1)
    x3 = x[:, None, :]
    y3 = y[:, None, :]
    z3 = z[:, None, :]
    d, t = _knn_dist(x3, y3, z3, center)
    t16 = jnp.broadcast_to(t.reshape(B * G, 1), (B * G, 16))
    cd, ci = _sc_compress(d.reshape(B * G * N), t16.reshape(B * G * 16),
                          B * G, N)
    idx = _topk_sel(cd.reshape(B, G, CAP), ci.reshape(B, G, CAP))
    flat_idx = (idx + jnp.arange(B, dtype=jnp.int32)[:, None, None] * N
                ).reshape(B * G * M)
    pts_pad = jnp.pad(pts.reshape(B * N, C), ((0, 0), (0, ROW_PAD - C)))
    cent_pad = jnp.pad(center.reshape(B * G, 3), ((0, 0), (0, ROW_PAD - 3)))
    rows = _sc_gather_normalize(flat_idx, pts_pad, cent_pad)
    neighborhood = rows[:, :C].reshape(B, G, M, C)
    return neighborhood, center


# speedup vs baseline: 1.0002x; 1.0002x over previous
import functools

import jax
import jax.numpy as jnp
from jax import lax
from jax.experimental import pallas as pl
from jax.experimental.pallas import tpu as pltpu
from jax.experimental.pallas import tpu_sc as plsc

NUM_GROUP_K = 512
GROUP_SIZE_K = 32
ROW_PAD = 16
GBLK = 64
NCHUNK = 32
CAP = 768
RCH = 8


def _fps_body(x_ref, y_ref, z_ref, cx_ref, cy_ref, cz_ref, dist_ref):
    B, N = x_ref.shape
    G = cx_ref.shape[1]
    x = x_ref[...]
    y = y_ref[...]
    z = z_ref[...]
    flane = lax.broadcasted_iota(jnp.int32, (B, N), 1).astype(jnp.float32)
    gcol = lax.broadcasted_iota(jnp.int32, (B, G), 1)
    bigf = jnp.float32(2.0 * N)

    dist_ref[...] = jnp.full((B, N), jnp.inf, dtype=jnp.float32)
    lx0 = x[:, 0:1]
    ly0 = y[:, 0:1]
    lz0 = z[:, 0:1]
    cx0 = jnp.where(gcol == 0, lx0, 0.0)
    cy0 = jnp.where(gcol == 0, ly0, 0.0)
    cz0 = jnp.where(gcol == 0, lz0, 0.0)

    def step(j, carry):
        lx, ly, lz, cx, cy, cz = carry
        dx = x - lx
        dy = y - ly
        dz = z - lz
        d = (dx * dx + dy * dy) + dz * dz
        dist = jnp.minimum(dist_ref[...], d)
        dist_ref[...] = dist
        mx = jnp.max(dist, axis=1, keepdims=True)
        nxt = jnp.min(jnp.where(dist == mx, flane, bigf), axis=1, keepdims=True)
        sel = flane == nxt
        lx = jnp.sum(jnp.where(sel, x, 0.0), axis=1, keepdims=True)
        ly = jnp.sum(jnp.where(sel, y, 0.0), axis=1, keepdims=True)
        lz = jnp.sum(jnp.where(sel, z, 0.0), axis=1, keepdims=True)
        hit = gcol == j
        cx = cx + jnp.where(hit, lx, 0.0)
        cy = cy + jnp.where(hit, ly, 0.0)
        cz = cz + jnp.where(hit, lz, 0.0)
        return lx, ly, lz, cx, cy, cz

    _, _, _, cx, cy, cz = lax.fori_loop(
        1, G, step, (lx0, ly0, lz0, cx0, cy0, cz0))
    cx_ref[...] = cx
    cy_ref[...] = cy
    cz_ref[...] = cz


def _fps_centers(x, y, z):
    B, N = x.shape
    G = NUM_GROUP_K
    out = jax.ShapeDtypeStruct((B, G), jnp.float32)
    return pl.pallas_call(
        _fps_body,
        out_shape=(out, out, out),
        scratch_shapes=[pltpu.VMEM((B, N), jnp.float32)],
    )(x, y, z)


def _dist_body(x_ref, y_ref, z_ref, c_ref, d_ref, t_ref):
    N = x_ref.shape[2]
    x = x_ref[0]
    y = y_ref[0]
    z = z_ref[0]
    c = c_ref[0]
    dx = c[:, 0:1] - x
    dy = c[:, 1:2] - y
    dz = c[:, 2:3] - z
    d0 = (dx * dx + dy * dy) + dz * dz
    d_ref[0] = d0
    cm = d0[:, 0:128]
    for ci in range(1, N // 128):
        cm = jnp.minimum(cm, d0[:, ci * 128:(ci + 1) * 128])
    cm = jnp.minimum(cm[:, 0:64], cm[:, 64:128])
    cm = jnp.minimum(cm[:, 0:NCHUNK], cm[:, NCHUNK:64])
    t_ref[0] = jnp.max(cm, axis=1, keepdims=True)


def _knn_dist(x3, y3, z3, center):
    B = x3.shape[0]
    N = x3.shape[2]
    G = NUM_GROUP_K
    grid = (B, G // GBLK)
    return pl.pallas_call(
        _dist_body,
        grid=grid,
        in_specs=[
            pl.BlockSpec((1, 1, N), lambda b, g: (b, 0, 0)),
            pl.BlockSpec((1, 1, N), lambda b, g: (b, 0, 0)),
            pl.BlockSpec((1, 1, N), lambda b, g: (b, 0, 0)),
            pl.BlockSpec((1, GBLK, 3), lambda b, g: (b, g, 0)),
        ],
        out_specs=(
            pl.BlockSpec((1, GBLK, N), lambda b, g: (b, g, 0)),
            pl.BlockSpec((1, GBLK, 1), lambda b, g: (b, g, 0)),
        ),
        out_shape=(
            jax.ShapeDtypeStruct((B, G, N), jnp.float32),
            jax.ShapeDtypeStruct((B, G, 1), jnp.float32),
        ),
    )(x3, y3, z3, center)


def _take16(x, idx):
    dn = lax.GatherDimensionNumbers(
        offset_dims=(), collapsed_slice_dims=(0,), start_index_map=(0,))
    return lax.gather(x, idx[:, None], dn, slice_sizes=(1,),
                      mode=lax.GatherScatterMode.PROMISE_IN_BOUNDS)


def _sc_compress(dflat, tflat, R2, N):
    info = plsc.get_sparse_core_info()
    nw = info.num_cores * info.num_subcores
    rpw = R2 // nw
    mesh = plsc.VectorSubcoreMesh(core_axis_name="c", subcore_axis_name="s")
    capb = CAP + 16

    @functools.partial(
        pl.kernel,
        mesh=mesh,
        compiler_params=pltpu.CompilerParams(use_tc_tiling_on_sc=False,
                                             needs_layout_passes=False),
        out_type=(
            jax.ShapeDtypeStruct((R2, CAP), jnp.float32),
            jax.ShapeDtypeStruct((R2, CAP), jnp.float32),
        ),
        scratch_types=[
            pltpu.VMEM((RCH * N,), jnp.float32),
            pltpu.VMEM((capb,), jnp.float32),
            pltpu.VMEM((capb,), jnp.float32),
            pltpu.VMEM((rpw * 16,), jnp.float32),
            pltpu.VMEM((16,), jnp.int32),
        ],
    )
    def k(d_hbm, t_hbm, cd_hbm, ci_hbm, rows_v, cd_v, ci_v, t_v, ptr_v):
        wid = lax.axis_index("s") * info.num_cores + lax.axis_index("c")
        rbase = wid * rpw
        pltpu.sync_copy(t_hbm.at[pl.ds(rbase * 16, rpw * 16)], t_v)
        iota16 = lax.iota(jnp.int32, 16)
        one16 = jnp.full((16,), 1, jnp.int32)
        zero16 = jnp.zeros((16,), jnp.int32)
        inf16 = jnp.full((16,), jnp.inf, jnp.float32)
        big16 = jnp.full((16,), jnp.float32(3 * N), jnp.float32)
        maxpos = jnp.full((16,), capb - 16, jnp.int32)
        lane15 = jnp.full((16,), 15, jnp.int32)

        def chunk_body(cix, _):
            pltpu.sync_copy(
                d_hbm.at[pl.ds((rbase + cix * RCH) * N, RCH * N)], rows_v)

            def row_body(rl, _):
                def pf(i, _):
                    cd_v[pl.ds(i * 16, 16)] = inf16
                    ci_v[pl.ds(i * 16, 16)] = big16
                    return 0
                lax.fori_loop(0, capb // 16, pf, 0)
                tvec = t_v[pl.ds((cix * RCH + rl) * 16, 16)]

                ptr_v[...] = jnp.zeros((16,), jnp.int32)

                def scan_body(i, _):
                    ptr = ptr_v[...]
                    for u in range(4):
                        iu = i * 4 + u
                        v = rows_v[pl.ds(rl * N + iu * 16, 16)]
                        mask = v <= tvec
                        cs = plsc.cumsum(jnp.where(mask, one16, zero16))
                        pos = jnp.minimum(ptr + cs - 1, maxpos)
                        posf = (iota16 + iu * 16).astype(jnp.float32)
                        plsc.store_scatter(cd_v, [pos], v, mask=mask)
                        plsc.store_scatter(ci_v, [pos], posf, mask=mask)
                        ptr = ptr + plsc.all_reduce_population_count(mask)
                    ptr_v[...] = ptr
                    return 0

                lax.fori_loop(0, N // 64, scan_body, 0)
                r = rbase + cix * RCH + rl
                pltpu.sync_copy(cd_v.at[pl.ds(0, CAP)], cd_hbm.at[r])
                pltpu.sync_copy(ci_v.at[pl.ds(0, CAP)], ci_hbm.at[r])
                return 0

            lax.fori_loop(0, RCH, row_body, 0)
            return 0

        lax.fori_loop(0, rpw // RCH, chunk_body, 0)

    return k(dflat, tflat)


def _sel_body(cd_ref, ci_ref, idx_ref, ds_ref):
    M = idx_ref.shape[2]
    d0 = cd_ref[0]
    idxf = ci_ref[0]
    ds_ref[...] = d0
    mcol = lax.broadcasted_iota(jnp.int32, (GBLK, M), 1)
    bigf = jnp.float32(3.0 * 8192)
    mn0 = jnp.min(d0, axis=1, keepdims=True)

    def body(j, carry):
        acc, mn = carry
        dcur = ds_ref[...]
        am = jnp.min(jnp.where(dcur == mn, idxf, bigf), axis=1, keepdims=True)
        dnew = jnp.where(idxf == am, jnp.inf, dcur)
        ds_ref[...] = dnew
        mn2 = jnp.min(dnew, axis=1, keepdims=True)
        return acc + jnp.where(mcol == j, am.astype(jnp.int32), 0), mn2

    acc, _ = lax.fori_loop(
        0, M, body, (jnp.zeros((GBLK, M), jnp.int32), mn0))
    idx_ref[0] = acc


def _topk_sel(cd3, ci3):
    B = cd3.shape[0]
    G = NUM_GROUP_K
    M = GROUP_SIZE_K
    grid = (B, G // GBLK)
    return pl.pallas_call(
        _sel_body,
        grid=grid,
        in_specs=[
            pl.BlockSpec((1, GBLK, CAP), lambda b, g: (b, g, 0)),
            pl.BlockSpec((1, GBLK, CAP), lambda b, g: (b, g, 0)),
        ],
        out_specs=pl.BlockSpec((1, GBLK, M), lambda b, g: (b, g, 0)),
        out_shape=jax.ShapeDtypeStruct((B, G, M), jnp.int32),
        scratch_shapes=[pltpu.VMEM((GBLK, CAP), jnp.float32)],
    )(cd3, ci3)


def _sc_gather_normalize(flat_idx, pts_pad, cent_pad):
    R = flat_idx.shape[0]
    info = plsc.get_sparse_core_info()
    nw = info.num_cores * info.num_subcores
    rpw = R // nw
    gpw = rpw // GROUP_SIZE_K
    mesh = plsc.VectorSubcoreMesh(core_axis_name="c", subcore_axis_name="s")

    @functools.partial(
        pl.kernel,
        mesh=mesh,
        compiler_params=pltpu.CompilerParams(use_tc_tiling_on_sc=False),
        out_type=jax.ShapeDtypeStruct((R, ROW_PAD), jnp.float32),
        scratch_types=[
            pltpu.VMEM((rpw,), jnp.int32),
            pltpu.VMEM((rpw, ROW_PAD), jnp.float32),
            pltpu.VMEM((gpw, ROW_PAD), jnp.float32),
            pltpu.SemaphoreType.DMA,
        ],
    )
    def k(idx_hbm, pts_hbm, cent_hbm, out_hbm, idx_v, rows_v, cent_v, sem):
        wid = lax.axis_index("s") * info.num_cores + lax.axis_index("c")
        rbase = wid * rpw
        pltpu.sync_copy(idx_hbm.at[pl.ds(rbase, rpw)], idx_v)
        pltpu.async_copy(pts_hbm.at[idx_v], rows_v, sem).wait()
        pltpu.sync_copy(cent_hbm.at[pl.ds(wid * gpw, gpw)], cent_v)

        def body(g, _):
            cvec = cent_v[g]
            base = g * GROUP_SIZE_K
            for j in range(GROUP_SIZE_K):
                rows_v[base + j] = rows_v[base + j] - cvec
            return 0

        lax.fori_loop(0, gpw, body, 0)
        pltpu.sync_copy(rows_v, out_hbm.at[pl.ds(rbase, rpw)])

    return k(flat_idx, pts_pad, cent_pad)


def kernel(pts):
    B, N, C = pts.shape
    G = NUM_GROUP_K
    M = GROUP_SIZE_K
    x = pts[:, :, 0]
    y = pts[:, :, 1]
    z = pts[:, :, 2]
    cx, cy, cz = _fps_centers(x, y, z)
    center = jnp.stack([cx, cy, cz], axis=-1)
    x3 = x[:, None, :]
    y3 = y[:, None, :]
    z3 = z[:, None, :]
    d, t = _knn_dist(x3, y3, z3, center)
    t16 = jnp.broadcast_to(t.reshape(B * G, 1), (B * G, 16))
    cd, ci = _sc_compress(d.reshape(B * G * N), t16.reshape(B * G * 16),
                          B * G, N)
    idx = _topk_sel(cd.reshape(B, G, CAP), ci.reshape(B, G, CAP))
    flat_idx = (idx + jnp.arange(B, dtype=jnp.int32)[:, None, None] * N
                ).reshape(B * G * M)
    pts_pad = jnp.pad(pts.reshape(B * N, C), ((0, 0), (0, ROW_PAD - C)))
    cent_pad = jnp.pad(center.reshape(B * G, 3), ((0, 0), (0, ROW_PAD - 3)))
    rows = _sc_gather_normalize(flat_idx, pts_pad, cent_pad)
    neighborhood = rows[:, :C].reshape(B, G, M, C)
    return neighborhood, center
